# baseline (speedup 1.0000x reference)
import jax
import jax.numpy as jnp
from jax import lax
from jax.experimental import pallas as pl
from jax.experimental.pallas import tpu as pltpu
from jax.experimental.pallas import tpu_sc as plsc


def _sc_gather(table, ids, batch, d_model):
    info = plsc.get_sparse_core_info()
    n_workers = info.num_cores * info.num_subcores
    b_per_w = batch // n_workers
    mesh = plsc.VectorSubcoreMesh(core_axis_name="c", subcore_axis_name="s")

    @pl.kernel(out_type=jax.ShapeDtypeStruct((batch, d_model), table.dtype),
               mesh=mesh,
               scratch_types=[
                   pltpu.VMEM((b_per_w,), jnp.int32),
                   pltpu.VMEM((b_per_w, d_model), table.dtype),
                   pltpu.SemaphoreType.DMA,
               ])
    def gather_kernel(tab_hbm, idx_hbm, out_hbm, idx_v, rows_v, sem):
        wid = lax.axis_index("s") * info.num_cores + lax.axis_index("c")
        base = wid * b_per_w
        pltpu.sync_copy(idx_hbm.at[pl.ds(base, b_per_w)], idx_v)
        pltpu.async_copy(tab_hbm.at[idx_v], rows_v, sem).wait()
        pltpu.sync_copy(rows_v, out_hbm.at[pl.ds(base, b_per_w)])

    return gather_kernel(table, ids)


def _tc_project_t(x, proj_w, proj_b_row, n_tile):
    batch, d_model = x.shape
    vocab = proj_w.shape[0]
    grid = (pl.cdiv(vocab, n_tile),)

    def mm_kernel(x_ref, w_ref, b_ref, o_ref):
        xb = x_ref[...].astype(jnp.bfloat16)
        wb = w_ref[...].astype(jnp.bfloat16)
        acc = jax.lax.dot_general(
            wb, xb,
            dimension_numbers=(((1,), (1,)), ((), ())),
            preferred_element_type=jnp.float32,
        )
        o_ref[...] = acc + jnp.transpose(b_ref[...])

    return pl.pallas_call(
        mm_kernel,
        grid=grid,
        in_specs=[
            pl.BlockSpec((batch, d_model), lambda i: (0, 0)),
            pl.BlockSpec((n_tile, d_model), lambda i: (i, 0)),
            pl.BlockSpec((1, n_tile), lambda i: (0, i)),
        ],
        out_specs=pl.BlockSpec((n_tile, batch), lambda i: (i, 0)),
        out_shape=jax.ShapeDtypeStruct((vocab, batch), jnp.float32),
        compiler_params=pltpu.CompilerParams(
            dimension_semantics=("parallel",),
        ),
    )(x, proj_w, proj_b_row)


def kernel(input_ids, embed_table, proj_w, proj_b):
    batch = input_ids.shape[0]
    d_model = embed_table.shape[1]
    ids = input_ids.astype(jnp.int32)
    x = _sc_gather(embed_table, ids, batch, d_model)
    logits_t = _tc_project_t(x, proj_w, proj_b.reshape(1, -1), n_tile=5504)
    return logits_t.T


# speedup vs baseline: 1.0097x; 1.0097x over previous
import jax
import jax.numpy as jnp
from jax import lax
from jax.experimental import pallas as pl
from jax.experimental.pallas import tpu as pltpu
from jax.experimental.pallas import tpu_sc as plsc


def _sc_gather(table, ids, batch, d_model):
    info = plsc.get_sparse_core_info()
    n_workers = 1 * info.num_subcores
    b_per_w = batch // n_workers
    mesh = plsc.VectorSubcoreMesh(core_axis_name="c", subcore_axis_name="s", num_cores=1)

    @pl.kernel(out_type=jax.ShapeDtypeStruct((batch, d_model), table.dtype),
               mesh=mesh,
               scratch_types=[
                   pltpu.VMEM((b_per_w,), jnp.int32),
                   pltpu.VMEM((b_per_w, d_model), table.dtype),
                   pltpu.SemaphoreType.DMA,
               ])
    def gather_kernel(tab_hbm, idx_hbm, out_hbm, idx_v, rows_v, sem):
        wid = lax.axis_index("s")
        base = wid * b_per_w
        pltpu.sync_copy(idx_hbm.at[pl.ds(base, b_per_w)], idx_v)
        pltpu.async_copy(tab_hbm.at[idx_v], rows_v, sem).wait()
        pltpu.sync_copy(rows_v, out_hbm.at[pl.ds(base, b_per_w)])

    return gather_kernel(table, ids)


def _tc_project_t(x, proj_w, proj_b_row, n_tile):
    batch, d_model = x.shape
    vocab = proj_w.shape[0]
    grid = (pl.cdiv(vocab, n_tile),)

    def mm_kernel(x_ref, w_ref, b_ref, o_ref):
        xb = x_ref[...].astype(jnp.bfloat16)
        wb = w_ref[...].astype(jnp.bfloat16)
        acc = jax.lax.dot_general(
            wb, xb,
            dimension_numbers=(((1,), (1,)), ((), ())),
            preferred_element_type=jnp.float32,
        )
        o_ref[...] = acc + jnp.transpose(b_ref[...])

    return pl.pallas_call(
        mm_kernel,
        grid=grid,
        in_specs=[
            pl.BlockSpec((batch, d_model), lambda i: (0, 0)),
            pl.BlockSpec((n_tile, d_model), lambda i: (i, 0)),
            pl.BlockSpec((1, n_tile), lambda i: (0, i)),
        ],
        out_specs=pl.BlockSpec((n_tile, batch), lambda i: (i, 0)),
        out_shape=jax.ShapeDtypeStruct((vocab, batch), jnp.float32),
        compiler_params=pltpu.CompilerParams(
            dimension_semantics=("parallel",),
        ),
    )(x, proj_w, proj_b_row)


def kernel(input_ids, embed_table, proj_w, proj_b):
    batch = input_ids.shape[0]
    d_model = embed_table.shape[1]
    ids = input_ids.astype(jnp.int32)
    x = _sc_gather(embed_table, ids, batch, d_model)
    logits_t = _tc_project_t(x, proj_w, proj_b.reshape(1, -1), n_tile=5120)
    return logits_t.T
